# baseline (speedup 1.0000x reference)
import functools

import jax
import jax.numpy as jnp
from jax.experimental import pallas as pl
from jax.experimental.pallas import tpu as pltpu
from jax.experimental.pallas import tpu_sc as plsc

NUM_ROWS = 15
DIM = 32

_mesh = plsc.ScalarSubcoreMesh(axis_name="c", num_cores=1)


@functools.partial(
    pl.kernel,
    out_type=jax.ShapeDtypeStruct((1, DIM), jnp.float32),
    mesh=_mesh,
    scratch_types=[
        pltpu.SMEM((1,), jnp.int32),
    ],
)
def _lookup(table_hbm, id_hbm, out_hbm, id_s):
    pltpu.sync_copy(id_hbm, id_s)
    i = id_s[0]
    pltpu.sync_copy(table_hbm.at[pl.ds(i, 1)], out_hbm)


def kernel(table, decision_id):
    out = _lookup(table, decision_id.reshape(1).astype(jnp.int32))
    return out.reshape(DIM)
